# baseline (device time: 10350 ns/iter reference)
import jax
import jax.numpy as jnp
from jax import lax
from jax.experimental import pallas as pl
from jax.experimental.pallas import tpu as pltpu

N_DEV = 4
CPS = 4
N_CHUNKS = 2 * CPS


def kernel(x):
    m, n = x.shape
    qm = m // N_CHUNKS

    def body(x_hbm, out_hbm, x_ref, fin_ref, recv_ref, pair_ref,
             in_sems, out_sems, send_sems, recv_sems):
        p = lax.axis_index("i")
        a = p ^ 1
        b = (N_DEV - 1) - p

        rows = [pl.ds(i * qm, qm) for i in range(N_CHUNKS)]

        fetch = [
            pltpu.make_async_copy(x_hbm.at[rows[i]], x_ref.at[rows[i]],
                                  in_sems.at[i])
            for i in range(N_CHUNKS)
        ]
        for f in fetch:
            f.start()

        barrier_sem = pltpu.get_barrier_semaphore()
        for nbr in [a, b]:
            pl.semaphore_signal(
                barrier_sem, inc=1,
                device_id=(nbr,), device_id_type=pl.DeviceIdType.MESH,
            )
        pl.semaphore_wait(barrier_sem, 2)

        def exchange(src, slot, dev):
            return pltpu.make_async_remote_copy(
                src_ref=src,
                dst_ref=recv_ref.at[slot],
                send_sem=send_sems.at[slot],
                recv_sem=recv_sems.at[slot],
                device_id=(dev,),
                device_id_type=pl.DeviceIdType.MESH,
            )

        ph1_dev = [a] * CPS + [b] * CPS
        ph2_dev = [b] * CPS + [a] * CPS
        order = [c + s * CPS for c in range(CPS) for s in (0, 1)]

        ph1 = [None] * N_CHUNKS
        for i in order:
            fetch[i].wait()
            ph1[i] = exchange(x_ref.at[rows[i]], i, ph1_dev[i])
            ph1[i].start()

        ph2 = [None] * N_CHUNKS
        for i in order:
            ph1[i].wait_recv()
            pair_ref[rows[i], :] = x_ref[rows[i], :] + recv_ref[i]
            ph2[i] = exchange(pair_ref.at[rows[i]], N_CHUNKS + i, ph2_dev[i])
            ph2[i].start()

        store = [None] * N_CHUNKS
        for i in order:
            ph2[i].wait_recv()
            fin_ref[rows[i], :] = pair_ref[rows[i], :] + recv_ref[N_CHUNKS + i]
            store[i] = pltpu.make_async_copy(
                fin_ref.at[rows[i]], out_hbm.at[rows[i]], out_sems.at[i]
            )
            store[i].start()

        for i in range(N_CHUNKS):
            ph1[i].wait_send()
            ph2[i].wait_send()
            store[i].wait()

    return pl.pallas_call(
        body,
        out_shape=jax.ShapeDtypeStruct((m, n), x.dtype),
        in_specs=[pl.BlockSpec(memory_space=pltpu.MemorySpace.HBM)],
        out_specs=pl.BlockSpec(memory_space=pltpu.MemorySpace.HBM),
        scratch_shapes=[
            pltpu.VMEM((m, n), x.dtype),
            pltpu.VMEM((m, n), x.dtype),
            pltpu.VMEM((2 * N_CHUNKS, qm, n), x.dtype),
            pltpu.VMEM((m, n), x.dtype),
            pltpu.SemaphoreType.DMA((N_CHUNKS,)),
            pltpu.SemaphoreType.DMA((N_CHUNKS,)),
            pltpu.SemaphoreType.DMA((2 * N_CHUNKS,)),
            pltpu.SemaphoreType.DMA((2 * N_CHUNKS,)),
        ],
        compiler_params=pltpu.CompilerParams(collective_id=0),
    )(x)


# device time: 9803 ns/iter; 1.0558x vs baseline; 1.0558x over previous
import jax
import jax.numpy as jnp
from jax import lax
from jax.experimental import pallas as pl
from jax.experimental.pallas import tpu as pltpu

N_DEV = 4
CPS = 4
N_CHUNKS = 2 * CPS


def kernel(x):
    m, n = x.shape
    qm = m // N_CHUNKS

    def body(x_ref, out_ref, recv_ref, pair_ref, send_sems, recv_sems):
        p = lax.axis_index("i")
        a = p ^ 1
        b = (N_DEV - 1) - p

        barrier_sem = pltpu.get_barrier_semaphore()
        for nbr in [a, b]:
            pl.semaphore_signal(
                barrier_sem, inc=1,
                device_id=(nbr,), device_id_type=pl.DeviceIdType.MESH,
            )
        pl.semaphore_wait(barrier_sem, 2)

        def exchange(src, slot, dev):
            return pltpu.make_async_remote_copy(
                src_ref=src,
                dst_ref=recv_ref.at[slot],
                send_sem=send_sems.at[slot],
                recv_sem=recv_sems.at[slot],
                device_id=(dev,),
                device_id_type=pl.DeviceIdType.MESH,
            )

        ph1_dev = [a] * CPS + [b] * CPS
        ph2_dev = [b] * CPS + [a] * CPS
        rows = [pl.ds(i * qm, qm) for i in range(N_CHUNKS)]
        order = [c + s * CPS for c in range(CPS) for s in (0, 1)]

        ph1 = [exchange(x_ref.at[rows[i]], i, ph1_dev[i]) for i in range(N_CHUNKS)]
        for i in range(N_CHUNKS):
            ph1[i].start()

        ph2 = [None] * N_CHUNKS
        for i in order:
            ph1[i].wait_recv()
            pair_ref[rows[i], :] = x_ref[rows[i], :] + recv_ref[i]
            ph2[i] = exchange(pair_ref.at[rows[i]], N_CHUNKS + i, ph2_dev[i])
            ph2[i].start()

        for i in order:
            ph2[i].wait_recv()
            out_ref[rows[i], :] = pair_ref[rows[i], :] + recv_ref[N_CHUNKS + i]

        for i in range(N_CHUNKS):
            ph1[i].wait_send()
            ph2[i].wait_send()

    return pl.pallas_call(
        body,
        out_shape=jax.ShapeDtypeStruct((m, n), x.dtype),
        in_specs=[pl.BlockSpec(memory_space=pltpu.VMEM)],
        out_specs=pl.BlockSpec(memory_space=pltpu.VMEM),
        scratch_shapes=[
            pltpu.VMEM((2 * N_CHUNKS, qm, n), x.dtype),
            pltpu.VMEM((m, n), x.dtype),
            pltpu.SemaphoreType.DMA((2 * N_CHUNKS,)),
            pltpu.SemaphoreType.DMA((2 * N_CHUNKS,)),
        ],
        compiler_params=pltpu.CompilerParams(collective_id=0),
    )(x)
